# baseline (speedup 1.0000x reference)
import functools

import jax
import jax.numpy as jnp
from jax import lax
from jax.experimental import pallas as pl
from jax.experimental.pallas import tpu as pltpu
from jax.experimental.pallas import tpu_sc as plsc

_VOCAB = 1000000
_HID = 64
_OUT = 2
_B = 4096
_L = 200

_NC = 2
_NS = 16
_NW = _NC * _NS
_IPW = _B // _NW
_C0 = 104
_C1 = _L - _C0


def _sc_pool(text_flat, emb_table):
    mesh = plsc.VectorSubcoreMesh(core_axis_name="c", subcore_axis_name="s")

    @functools.partial(
        pl.kernel,
        mesh=mesh,
        compiler_params=pltpu.CompilerParams(use_tc_tiling_on_sc=False),
        out_type=jax.ShapeDtypeStruct((_B, _HID), jnp.float32),
        scratch_types=[
            pltpu.VMEM((_IPW * _L,), jnp.int32),
            pltpu.VMEM((_L, _HID), jnp.float32),
            pltpu.VMEM((_IPW, _HID), jnp.float32),
            pltpu.SemaphoreType.DMA,
        ],
    )
    def k(text_hbm, table_hbm, out_hbm, idx_v, rows_v, pooled_v, sem):
        wid = lax.axis_index("s") * _NC + lax.axis_index("c")
        ibase = wid * (_IPW * _L)
        pltpu.sync_copy(text_hbm.at[pl.ds(ibase, _IPW * _L)], idx_v)

        def item(i, carry):
            off = i * _L
            cp0 = pltpu.async_copy(
                table_hbm.at[idx_v.at[pl.ds(off, _C0)]],
                rows_v.at[pl.ds(0, _C0)], sem)
            cp1 = pltpu.async_copy(
                table_hbm.at[idx_v.at[pl.ds(off + _C0, _C1)]],
                rows_v.at[pl.ds(_C0, _C1)], sem)
            cp0.wait()
            cp1.wait()

            def red(l, accs):
                return tuple(accs[c] + rows_v[l, pl.ds(c * 16, 16)]
                             for c in range(4))

            accs = tuple(jnp.zeros((16,), jnp.float32) for _ in range(4))
            accs = lax.fori_loop(0, _L, red, accs)
            for c in range(4):
                pooled_v[i, pl.ds(c * 16, 16)] = accs[c]
            return carry

        lax.fori_loop(0, _IPW, item, 0)
        pltpu.sync_copy(pooled_v, out_hbm.at[pl.ds(wid * _IPW, _IPW)])

    return k(text_flat, emb_table)


def _tc_head(pooled, wt, bias):
    def body(x_ref, w_ref, b_ref, o_ref):
        x = x_ref[...] * (1.0 / _L)
        o_ref[...] = (
            jnp.dot(x, w_ref[...], preferred_element_type=jnp.float32)
            + b_ref[...]
        )

    return pl.pallas_call(
        body,
        out_shape=jax.ShapeDtypeStruct((_B, _OUT), jnp.float32),
    )(pooled, wt, bias)


def kernel(text, emb_table, fc1_w, fc1_b):
    pooled = _sc_pool(text.reshape(-1), emb_table)
    return _tc_head(pooled, fc1_w.T, fc1_b.reshape(1, _OUT))


# speedup vs baseline: 1.1749x; 1.1749x over previous
import functools

import jax
import jax.numpy as jnp
from jax import lax
from jax.experimental import pallas as pl
from jax.experimental.pallas import tpu as pltpu
from jax.experimental.pallas import tpu_sc as plsc

_VOCAB = 1000000
_HID = 64
_OUT = 2
_B = 4096
_L = 200

_NC = 2
_NS = 16
_NW = _NC * _NS
_IPW = _B // _NW
_C0 = 104
_C1 = _L - _C0


_UNROLL = 8


def _sc_pool(text_flat, emb_table):
    mesh = plsc.VectorSubcoreMesh(core_axis_name="c", subcore_axis_name="s")

    @functools.partial(
        pl.kernel,
        mesh=mesh,
        compiler_params=pltpu.CompilerParams(use_tc_tiling_on_sc=False),
        out_type=jax.ShapeDtypeStruct((_B, _HID), jnp.float32),
        scratch_types=[
            pltpu.VMEM((_IPW * _L,), jnp.int32),
            pltpu.VMEM((_L, _HID), jnp.float32),
            pltpu.VMEM((_L, _HID), jnp.float32),
            pltpu.VMEM((_IPW, _HID), jnp.float32),
            pltpu.SemaphoreType.DMA,
            pltpu.SemaphoreType.DMA,
        ],
    )
    def k(text_hbm, table_hbm, out_hbm, idx_v, rows0, rows1, pooled_v,
          sem0, sem1):
        wid = lax.axis_index("s") * _NC + lax.axis_index("c")
        ibase = wid * (_IPW * _L)
        pltpu.sync_copy(text_hbm.at[pl.ds(ibase, _IPW * _L)], idx_v)

        def gather(i, buf, sem):
            off = i * _L
            pltpu.async_copy(
                table_hbm.at[idx_v.at[pl.ds(off, _C0)]],
                buf.at[pl.ds(0, _C0)], sem)
            pltpu.async_copy(
                table_hbm.at[idx_v.at[pl.ds(off + _C0, _C1)]],
                buf.at[pl.ds(_C0, _C1)], sem)

        def wait_gather(i, buf, sem):
            off = i * _L
            pltpu.make_async_copy(
                table_hbm.at[idx_v.at[pl.ds(off, _C0)]],
                buf.at[pl.ds(0, _C0)], sem).wait()
            pltpu.make_async_copy(
                table_hbm.at[idx_v.at[pl.ds(off + _C0, _C1)]],
                buf.at[pl.ds(_C0, _C1)], sem).wait()

        def reduce_to(i, buf):
            def step(s, accs):
                a = list(accs)
                for r in range(_UNROLL):
                    l = s * _UNROLL + r
                    for c in range(4):
                        a[c] = a[c] + buf[l, pl.ds(c * 16, 16)]
                return tuple(a)

            accs = lax.fori_loop(
                0, _L // _UNROLL, step,
                tuple(jnp.zeros((16,), jnp.float32) for _ in range(4)))
            for c in range(4):
                pooled_v[i, pl.ds(c * 16, 16)] = accs[c]

        gather(0, rows0, sem0)

        @pl.loop(0, _IPW, step=2)
        def _(i):
            gather(i + 1, rows1, sem1)
            wait_gather(i, rows0, sem0)
            reduce_to(i, rows0)

            @pl.when(i + 2 < _IPW)
            def _():
                gather(i + 2, rows0, sem0)

            wait_gather(i + 1, rows1, sem1)
            reduce_to(i + 1, rows1)

        pltpu.sync_copy(pooled_v, out_hbm.at[pl.ds(wid * _IPW, _IPW)])

    return k(text_flat, emb_table)


def _tc_head(pooled, wt, bias):
    def body(x_ref, w_ref, b_ref, o_ref):
        x = x_ref[...] * (1.0 / _L)
        o_ref[...] = (
            jnp.dot(x, w_ref[...], preferred_element_type=jnp.float32)
            + b_ref[...]
        )

    return pl.pallas_call(
        body,
        out_shape=jax.ShapeDtypeStruct((_B, _OUT), jnp.float32),
    )(pooled, wt, bias)


def kernel(text, emb_table, fc1_w, fc1_b):
    pooled = _sc_pool(text.reshape(-1), emb_table)
    return _tc_head(pooled, fc1_w.T, fc1_b.reshape(1, _OUT))
